# baseline (speedup 1.0000x reference)
import dataclasses
import functools

import jax
import jax.numpy as jnp
from jax import lax
from jax.experimental import pallas as pl
from jax.experimental.pallas import tpu as pltpu
from jax.experimental.pallas import tpu_sc as plsc

TAU = 2.0
V_TH = 1.0
BN_EPS = 1e-5
EPS = 1e-09
CAP_FACTOR = 1.25
MIN_EXPERT_CAPACITY = 4
E = 16


def _sc_compiler_params():
    cp = pltpu.CompilerParams()
    if "needs_layout_passes" in pltpu.CompilerParams.__dataclass_fields__:
        cp = dataclasses.replace(cp, needs_layout_passes=False)
    return cp


def _stage_a_body(x_ref, w_ref, b_ref, g_ref, bt_ref, lg_ref, bnss_ref,
                  s_ref, sq_ref, *, T, C, N, total_tokens):
    b = pl.program_id(0)
    nb = pl.num_programs(0)

    @pl.when(b == 0)
    def _():
        s_ref[...] = jnp.zeros_like(s_ref)
        sq_ref[...] = jnp.zeros_like(sq_ref)

    w = w_ref[...]
    bias = b_ref[...]
    v = jnp.zeros((N, C), jnp.float32)
    acc = jnp.zeros((1, E), jnp.float32)
    accsq = jnp.zeros((1, E), jnp.float32)
    for t in range(T):
        v = (v + x_ref[t, 0]) * 0.5
        s = (v >= V_TH).astype(jnp.float32)
        v = v * (1.0 - s)
        lg = jax.lax.dot_general(s, w, (((1,), (1,)), ((), ())),
                                 preferred_element_type=jnp.float32) + bias
        lg_ref[t, 0] = lg
        acc = acc + jnp.sum(lg, axis=0, keepdims=True)
        accsq = accsq + jnp.sum(lg * lg, axis=0, keepdims=True)
    s_ref[...] += acc
    sq_ref[...] += accsq

    @pl.when(b == nb - 1)
    def _():
        inv_n = 1.0 / float(total_tokens)
        mean = s_ref[...] * inv_n
        var = sq_ref[...] * inv_n - mean * mean
        scale = g_ref[...] * jax.lax.rsqrt(var + BN_EPS)
        shift = bt_ref[...] - mean * scale
        bnss_ref[0:1, :] = scale
        bnss_ref[1:2, :] = shift


def _router_body(lg_hbm, bnss_hbm, e1_hbm, p1_hbm, g1_hbm, e2_hbm, p2_hbm,
                 g2_hbm, proxy_hbm, c1f_hbm,
                 slab, bnss_v, e1b, p1b, g1b, e2b, p2pb, g2nb, p2b, g2b,
                 proxyb, c1fb, c1capb, sem, *, N, cap):
    cid = lax.axis_index("c")
    sid = lax.axis_index("s")
    g = cid * 16 + sid
    capf = float(cap)

    pltpu.async_copy(lg_hbm.at[g], slab, sem).wait()
    pltpu.async_copy(bnss_hbm, bnss_v, sem).wait()

    scale = bnss_v[0]
    shift = bnss_v[1]
    iota = lax.iota(jnp.int32, E)
    iotaf = iota.astype(jnp.float32)
    one = jnp.float32(1.0)

    def block(i, carry):
        c1, c2, proxy = carry
        zero16 = jnp.zeros((E,), jnp.float32)
        e1a = p1a = g1a = e2a = p2pa = g2na = zero16
        for k in range(E):
            y = slab[pl.ds(i * (E * E) + k * E, E)] * scale + shift
            mx = jnp.max(y, axis=0)
            ex = jnp.exp(y - mx)
            ssum = jnp.sum(ex, axis=0)
            p = ex / jnp.full((E,), ssum, jnp.float32)
            m1 = jnp.max(p, axis=0)
            eq1 = (p == m1).astype(jnp.float32)
            mask1 = eq1 * (plsc.cumsum(eq1) == one).astype(jnp.float32)
            p2 = p * (one - mask1)
            m2 = jnp.max(p2, axis=0)
            eq2 = (p2 == m2).astype(jnp.float32)
            mask2 = eq2 * (plsc.cumsum(eq2) == one).astype(jnp.float32)

            denv = jnp.full((E,), m1 + m2 + EPS, jnp.float32)
            g1v = jnp.full((E,), m1, jnp.float32) / denv
            g2v = jnp.full((E,), m2, jnp.float32) / denv
            keep1 = mask1 * (c1 < capf).astype(jnp.float32)
            kept1 = jnp.sum(keep1, axis=0)
            pos1 = jnp.sum(keep1 * c1, axis=0)

            lk = iota == k
            e1a = jnp.where(lk, jnp.full((E,), jnp.sum(mask1 * iotaf)), e1a)
            p1a = jnp.where(
                lk, jnp.full((E,), jnp.where(kept1 > 0, pos1, -1.0)), p1a)
            g1a = jnp.where(lk, g1v * jnp.full((E,), kept1), g1a)
            e2a = jnp.where(lk, jnp.full((E,), jnp.sum(mask2 * iotaf)), e2a)
            p2pa = jnp.where(lk, jnp.full((E,), jnp.sum(mask2 * c2)), p2pa)
            g2na = jnp.where(lk, g2v, g2na)

            c1 = c1 + mask1
            c2 = c2 + mask2
            proxy = proxy + p
        sl = pl.ds(i * E, E)
        e1b[sl] = e1a
        p1b[sl] = p1a
        g1b[sl] = g1a
        e2b[sl] = e2a
        p2pb[sl] = p2pa
        g2nb[sl] = g2na
        return c1, c2, proxy

    zero16 = jnp.zeros((E,), jnp.float32)
    c1, c2, proxy = lax.fori_loop(0, N // E, block,
                                  (zero16, zero16, zero16))
    c1fb[...] = c1
    proxyb[...] = proxy
    c1capb[...] = jnp.minimum(c1, capf)

    def pass2(i, carry):
        sl = pl.ds(i * E, E)
        base = plsc.load_gather(c1capb, [e2b[sl].astype(jnp.int32)])
        pos2 = p2pb[sl] + base
        ok = pos2 < capf
        p2b[sl] = jnp.where(ok, pos2, -1.0)
        g2b[sl] = jnp.where(ok, g2nb[sl], 0.0)
        return carry

    lax.fori_loop(0, N // E, pass2, 0)

    pltpu.async_copy(e1b, e1_hbm.at[g], sem).wait()
    pltpu.async_copy(p1b, p1_hbm.at[g], sem).wait()
    pltpu.async_copy(g1b, g1_hbm.at[g], sem).wait()
    pltpu.async_copy(e2b, e2_hbm.at[g], sem).wait()
    pltpu.async_copy(p2b, p2_hbm.at[g], sem).wait()
    pltpu.async_copy(g2b, g2_hbm.at[g], sem).wait()
    pltpu.async_copy(proxyb, proxy_hbm.at[g], sem).wait()
    pltpu.async_copy(c1fb, c1f_hbm.at[g], sem).wait()


def _stage_b_body(e1_ref, p1_ref, g1_ref, e2_ref, p2_ref, g2_ref,
                  proxy_ref, c1f_ref, comb_ref, disp_ref, loss_ref,
                  lacc_ref, *, N, cap):
    g = pl.program_id(0)
    ng = pl.num_programs(0)

    @pl.when(g == 0)
    def _():
        lacc_ref[...] = jnp.zeros_like(lacc_ref)

    e1 = e1_ref[0]
    p1 = p1_ref[0]
    g1 = g1_ref[0]
    e2 = e2_ref[0]
    p2 = p2_ref[0]
    g2 = g2_ref[0]

    iota_e = jax.lax.broadcasted_iota(jnp.int32, (1, E, N), 1).astype(
        jnp.float32)
    iota_p = jax.lax.broadcasted_iota(jnp.int32, (cap, 1, N), 0).astype(
        jnp.float32)
    me1 = (iota_e == e1[:, None, :]).astype(jnp.float32)
    me2 = (iota_e == e2[:, None, :]).astype(jnp.float32)
    oh1 = (iota_p == p1[None]).astype(jnp.float32)
    oh2 = (iota_p == p2[None]).astype(jnp.float32)
    d1 = me1 * oh1
    d2 = me2 * oh2
    comb_ref[0] = d1 * g1[:, None, :] + d2 * g2[:, None, :]
    disp_ref[0] = d1 + d2

    lacc_ref[...] += proxy_ref[0] * c1f_ref[0]

    @pl.when(g == ng - 1)
    def _():
        scale_l = float(E * E) / (float(N) * float(N) * float(ng) * float(E))
        loss_ref[...] = jnp.sum(
            lacc_ref[...], axis=1, keepdims=True) * scale_l


def kernel(x, conv_w, conv_b, bn_gamma, bn_beta):
    T, B, C, H, W = x.shape
    N = H * W
    G = T * B
    cap = min(N, int(N * CAP_FACTOR / E))
    cap = max(cap, MIN_EXPERT_CAPACITY)

    xt = x.transpose(0, 1, 3, 4, 2).reshape(T, B, N, C)

    stage_a = pl.pallas_call(
        functools.partial(_stage_a_body, T=T, C=C, N=N, total_tokens=G * N),
        grid=(B,),
        in_specs=[
            pl.BlockSpec((T, 1, N, C), lambda b: (0, b, 0, 0)),
            pl.BlockSpec((E, C), lambda b: (0, 0)),
            pl.BlockSpec((1, E), lambda b: (0, 0)),
            pl.BlockSpec((1, E), lambda b: (0, 0)),
            pl.BlockSpec((1, E), lambda b: (0, 0)),
        ],
        out_specs=[
            pl.BlockSpec((T, 1, N, E), lambda b: (0, b, 0, 0)),
            pl.BlockSpec((2, E), lambda b: (0, 0)),
        ],
        out_shape=[
            jax.ShapeDtypeStruct((T, B, N, E), jnp.float32),
            jax.ShapeDtypeStruct((2, E), jnp.float32),
        ],
        scratch_shapes=[
            pltpu.VMEM((1, E), jnp.float32),
            pltpu.VMEM((1, E), jnp.float32),
        ],
    )
    lg, bnss = stage_a(xt, conv_w, conv_b.reshape(1, E),
                       bn_gamma.reshape(1, E), bn_beta.reshape(1, E))

    lgt = lg.reshape(G, N * E)

    rowf = jax.ShapeDtypeStruct((G, N), jnp.float32)
    router = pl.kernel(
        functools.partial(_router_body, N=N, cap=cap),
        out_type=[rowf, rowf, rowf, rowf, rowf, rowf,
                  jax.ShapeDtypeStruct((G, E), jnp.float32),
                  jax.ShapeDtypeStruct((G, E), jnp.float32)],
        mesh=plsc.VectorSubcoreMesh(core_axis_name="c", subcore_axis_name="s"),
        compiler_params=_sc_compiler_params(),
        scratch_types=[
            pltpu.VMEM((N * E,), jnp.float32),
            pltpu.VMEM((2, E), jnp.float32),
            pltpu.VMEM((N,), jnp.float32),
            pltpu.VMEM((N,), jnp.float32),
            pltpu.VMEM((N,), jnp.float32),
            pltpu.VMEM((N,), jnp.float32),
            pltpu.VMEM((N,), jnp.float32),
            pltpu.VMEM((N,), jnp.float32),
            pltpu.VMEM((N,), jnp.float32),
            pltpu.VMEM((N,), jnp.float32),
            pltpu.VMEM((E,), jnp.float32),
            pltpu.VMEM((E,), jnp.float32),
            pltpu.VMEM((E,), jnp.float32),
            pltpu.SemaphoreType.DMA,
        ],
    )
    e1, p1, g1, e2, p2, g2, proxy, c1f = router(lgt, bnss)

    stage_b = pl.pallas_call(
        functools.partial(_stage_b_body, N=N, cap=cap),
        grid=(G,),
        in_specs=[pl.BlockSpec((1, 1, N), lambda g: (g, 0, 0))] * 6
        + [pl.BlockSpec((1, 1, E), lambda g: (g, 0, 0))] * 2,
        out_specs=[
            pl.BlockSpec((1, cap, E, N), lambda g: (g, 0, 0, 0)),
            pl.BlockSpec((1, cap, E, N), lambda g: (g, 0, 0, 0)),
            pl.BlockSpec((1, 1), lambda g: (0, 0)),
        ],
        out_shape=[
            jax.ShapeDtypeStruct((G, cap, E, N), jnp.float32),
            jax.ShapeDtypeStruct((G, cap, E, N), jnp.float32),
            jax.ShapeDtypeStruct((1, 1), jnp.float32),
        ],
        scratch_shapes=[
            pltpu.VMEM((1, E), jnp.float32),
        ],
    )
    comb, disp, loss = stage_b(
        e1.reshape(G, 1, N), p1.reshape(G, 1, N), g1.reshape(G, 1, N),
        e2.reshape(G, 1, N), p2.reshape(G, 1, N), g2.reshape(G, 1, N),
        proxy.reshape(G, 1, E), c1f.reshape(G, 1, E))

    disp = disp.transpose(0, 3, 2, 1)
    comb = comb.transpose(0, 3, 2, 1)
    return disp, comb, loss.reshape(()), cap
